# baseline (speedup 1.0000x reference)
import functools

import jax
import jax.numpy as jnp
from jax import lax
from jax.experimental import pallas as pl
from jax.experimental.pallas import tpu as pltpu
from jax.experimental.pallas import tpu_sc as plsc

_LANES = 16


def _sc_base_body(toks_per_w, n_used, stab_id_hbm, cycle_id_hbm, stab_hbm,
                  cyc_hbm, base_hbm, sidx, cidx, srows, crows, sem):
    D = srows.shape[1]
    wid = lax.axis_index("s") * 2 + lax.axis_index("c")

    @pl.when(wid < n_used)
    def _():
        tok0 = wid * toks_per_w
        cpi1 = pltpu.async_copy(stab_id_hbm.at[pl.ds(tok0, toks_per_w)],
                                sidx, sem)
        cpi2 = pltpu.async_copy(cycle_id_hbm.at[pl.ds(tok0, toks_per_w)],
                                cidx, sem)
        cpi1.wait()
        cpi2.wait()
        cpg1 = pltpu.async_copy(stab_hbm.at[sidx], srows, sem)
        cpg2 = pltpu.async_copy(cyc_hbm.at[cidx], crows, sem)
        cpg1.wait()
        cpg2.wait()
        for j in range(toks_per_w):
            for k in range(D // _LANES):
                ds = pl.ds(k * _LANES, _LANES)
                srows[j, ds] = srows[j, ds] + crows[j, ds]
        pltpu.sync_copy(srows, base_hbm.at[pl.ds(tok0, toks_per_w)])


def _sc_build_base(stab_id, cycle_id, stab_table, cycle_table):
    L = stab_id.shape[0]
    D = stab_table.shape[1]
    toks_per_w = 8
    n_used = L // toks_per_w
    mesh = plsc.VectorSubcoreMesh(core_axis_name="c", subcore_axis_name="s")
    body = functools.partial(_sc_base_body, toks_per_w, n_used)
    return pl.kernel(
        body,
        out_type=jax.ShapeDtypeStruct((L, D), jnp.float32),
        mesh=mesh,
        scratch_types=[
            pltpu.VMEM((toks_per_w,), jnp.int32),
            pltpu.VMEM((toks_per_w,), jnp.int32),
            pltpu.VMEM((toks_per_w, D), jnp.float32),
            pltpu.VMEM((toks_per_w, D), jnp.float32),
            pltpu.SemaphoreType.DMA,
        ],
    )(stab_id, cycle_id, stab_table, cycle_table)


def _tc_stream_body(syn_ref, base_ref, val_ref, out_ref):
    syn = syn_ref[...].astype(jnp.float32)
    b0 = base_ref[...] + val_ref[0, :][None, :]
    diff = val_ref[1, :] - val_ref[0, :]
    out_ref[...] = b0[None, :, :] + syn[:, :, None] * diff[None, None, :]


def kernel(syndrome, stab_id, cycle_id, stab_table, cycle_table, val_table):
    B, L = syndrome.shape
    D = stab_table.shape[1]
    BB = 16

    base = _sc_build_base(stab_id.astype(jnp.int32),
                          cycle_id.astype(jnp.int32),
                          stab_table, cycle_table)
    syn = syndrome.astype(jnp.int32)

    return pl.pallas_call(
        _tc_stream_body,
        grid=(B // BB,),
        in_specs=[
            pl.BlockSpec((BB, L), lambda i: (i, 0)),
            pl.BlockSpec((L, D), lambda i: (0, 0)),
            pl.BlockSpec((2, D), lambda i: (0, 0)),
        ],
        out_specs=pl.BlockSpec((BB, L, D), lambda i: (i, 0, 0)),
        out_shape=jax.ShapeDtypeStruct((B, L, D), jnp.float32),
    )(syn, base, val_table)


# speedup vs baseline: 1.0529x; 1.0529x over previous
import functools

import jax
import jax.numpy as jnp
from jax import lax
from jax.experimental import pallas as pl
from jax.experimental.pallas import tpu as pltpu
from jax.experimental.pallas import tpu_sc as plsc

_LANES = 16


def _sc_base_body(toks_per_w, n_used, stab_id_hbm, cycle_id_hbm, stab_hbm,
                  cyc_hbm, base_hbm, sidx, cidx, srows, crows, sem):
    D = srows.shape[1]
    wid = lax.axis_index("s") * 2 + lax.axis_index("c")

    @pl.when(wid < n_used)
    def _():
        tok0 = wid * toks_per_w
        cpi1 = pltpu.async_copy(stab_id_hbm.at[pl.ds(tok0, toks_per_w)],
                                sidx, sem)
        cpi2 = pltpu.async_copy(cycle_id_hbm.at[pl.ds(tok0, toks_per_w)],
                                cidx, sem)
        cpi1.wait()
        cpi2.wait()
        cpg1 = pltpu.async_copy(stab_hbm.at[sidx], srows, sem)
        cpg2 = pltpu.async_copy(cyc_hbm.at[cidx], crows, sem)
        cpg1.wait()
        cpg2.wait()
        for j in range(toks_per_w):
            for k in range(D // _LANES):
                ds = pl.ds(k * _LANES, _LANES)
                srows[j, ds] = srows[j, ds] + crows[j, ds]
        pltpu.sync_copy(srows, base_hbm.at[pl.ds(tok0, toks_per_w)])


def _sc_build_base(stab_id, cycle_id, stab_table, cycle_table):
    L = stab_id.shape[0]
    D = stab_table.shape[1]
    toks_per_w = 8
    n_used = L // toks_per_w
    mesh = plsc.VectorSubcoreMesh(core_axis_name="c", subcore_axis_name="s")
    body = functools.partial(_sc_base_body, toks_per_w, n_used)
    return pl.kernel(
        body,
        out_type=jax.ShapeDtypeStruct((L, D), jnp.float32),
        mesh=mesh,
        scratch_types=[
            pltpu.VMEM((toks_per_w,), jnp.int32),
            pltpu.VMEM((toks_per_w,), jnp.int32),
            pltpu.VMEM((toks_per_w, D), jnp.float32),
            pltpu.VMEM((toks_per_w, D), jnp.float32),
            pltpu.SemaphoreType.DMA,
        ],
    )(stab_id, cycle_id, stab_table, cycle_table)


def _tc_selfbase_body(sid_ref, cid_ref, syn_ref, stab_ref, cyc_ref, val_ref,
                      out_ref, base_ref):
    L, D = base_ref.shape

    @pl.when(pl.program_id(0) == 0)
    def _compute_base():
        ns = stab_ref.shape[0]
        nc = cyc_ref.shape[0]
        oh_s = (sid_ref[...] == lax.broadcasted_iota(jnp.int32, (L, ns), 1)
                ).astype(jnp.float32)
        oh_c = (cid_ref[...] == lax.broadcasted_iota(jnp.int32, (L, nc), 1)
                ).astype(jnp.float32)
        b = jnp.dot(oh_s, stab_ref[...], preferred_element_type=jnp.float32)
        b += jnp.dot(oh_c, cyc_ref[...], preferred_element_type=jnp.float32)
        base_ref[...] = b + val_ref[0, :][None, :]

    syn = syn_ref[...].astype(jnp.float32)
    diff = val_ref[1, :] - val_ref[0, :]
    out_ref[...] = (base_ref[...][None, :, :]
                    + syn[:, :, None] * diff[None, None, :])


def _tc_scbase_body(syn_ref, base_ref, val_ref, part_ref, out_ref):
    del part_ref
    syn = syn_ref[...].astype(jnp.float32)
    b0 = base_ref[...] + val_ref[0, :][None, :]
    diff = val_ref[1, :] - val_ref[0, :]
    out_ref[...] = b0[None, :, :] + syn[:, :, None] * diff[None, None, :]


def kernel(syndrome, stab_id, cycle_id, stab_table, cycle_table, val_table):
    B, L = syndrome.shape
    D = stab_table.shape[1]
    BB = 32
    B1 = 256
    nb1 = B1 // BB

    sid32 = stab_id.astype(jnp.int32)
    cid32 = cycle_id.astype(jnp.int32)
    syn = syndrome.astype(jnp.int32)

    base = _sc_build_base(sid32, cid32, stab_table, cycle_table)

    ns = stab_table.shape[0]
    ns_pad = ((ns + 127) // 128) * 128
    stab_pad = (jnp.pad(stab_table, ((0, ns_pad - ns), (0, 0)))
                if ns_pad != ns else stab_table)

    part = pl.pallas_call(
        _tc_selfbase_body,
        grid=(nb1,),
        in_specs=[
            pl.BlockSpec((L, 1), lambda i: (0, 0)),
            pl.BlockSpec((L, 1), lambda i: (0, 0)),
            pl.BlockSpec((BB, L), lambda i: (i, 0)),
            pl.BlockSpec((ns_pad, D), lambda i: (0, 0)),
            pl.BlockSpec(cycle_table.shape, lambda i: (0, 0)),
            pl.BlockSpec((2, D), lambda i: (0, 0)),
        ],
        out_specs=pl.BlockSpec((BB, L, D), lambda i: (i, 0, 0)),
        out_shape=jax.ShapeDtypeStruct((B, L, D), jnp.float32),
        scratch_shapes=[pltpu.VMEM((L, D), jnp.float32)],
    )(sid32.reshape(L, 1), cid32.reshape(L, 1), syn, stab_pad, cycle_table,
      val_table)

    return pl.pallas_call(
        _tc_scbase_body,
        grid=((B - B1) // BB,),
        in_specs=[
            pl.BlockSpec((BB, L), lambda i: (i + nb1, 0)),
            pl.BlockSpec((L, D), lambda i: (0, 0)),
            pl.BlockSpec((2, D), lambda i: (0, 0)),
            pl.BlockSpec(memory_space=pl.ANY),
        ],
        out_specs=pl.BlockSpec((BB, L, D), lambda i: (i + nb1, 0, 0)),
        out_shape=jax.ShapeDtypeStruct((B, L, D), jnp.float32),
        input_output_aliases={3: 0},
    )(syn, base, val_table, part)


# speedup vs baseline: 1.0537x; 1.0007x over previous
import functools

import jax
import jax.numpy as jnp
from jax import lax
from jax.experimental import pallas as pl
from jax.experimental.pallas import tpu as pltpu
from jax.experimental.pallas import tpu_sc as plsc

_LANES = 16


def _sc_base_body(toks_per_w, n_used, stab_id_hbm, cycle_id_hbm, stab_hbm,
                  cyc_hbm, base_hbm, sidx, cidx, srows, crows, sem):
    D = srows.shape[1]
    wid = lax.axis_index("s") * 2 + lax.axis_index("c")

    @pl.when(wid < n_used)
    def _():
        tok0 = wid * toks_per_w
        cpi1 = pltpu.async_copy(stab_id_hbm.at[pl.ds(tok0, toks_per_w)],
                                sidx, sem)
        cpi2 = pltpu.async_copy(cycle_id_hbm.at[pl.ds(tok0, toks_per_w)],
                                cidx, sem)
        cpi1.wait()
        cpi2.wait()
        cpg1 = pltpu.async_copy(stab_hbm.at[sidx], srows, sem)
        cpg2 = pltpu.async_copy(cyc_hbm.at[cidx], crows, sem)
        cpg1.wait()
        cpg2.wait()
        for j in range(toks_per_w):
            for k in range(D // _LANES):
                ds = pl.ds(k * _LANES, _LANES)
                srows[j, ds] = srows[j, ds] + crows[j, ds]
        pltpu.sync_copy(srows, base_hbm.at[pl.ds(tok0, toks_per_w)])


def _sc_build_base(stab_id, cycle_id, stab_table, cycle_table):
    L = stab_id.shape[0]
    D = stab_table.shape[1]
    toks_per_w = 8
    n_used = L // toks_per_w
    mesh = plsc.VectorSubcoreMesh(core_axis_name="c", subcore_axis_name="s")
    body = functools.partial(_sc_base_body, toks_per_w, n_used)
    return pl.kernel(
        body,
        out_type=jax.ShapeDtypeStruct((L, D), jnp.float32),
        mesh=mesh,
        scratch_types=[
            pltpu.VMEM((toks_per_w,), jnp.int32),
            pltpu.VMEM((toks_per_w,), jnp.int32),
            pltpu.VMEM((toks_per_w, D), jnp.float32),
            pltpu.VMEM((toks_per_w, D), jnp.float32),
            pltpu.SemaphoreType.DMA,
        ],
    )(stab_id, cycle_id, stab_table, cycle_table)


def _tc_selfbase_body(sid_ref, cid_ref, syn_ref, stab_ref, cyc_ref, val_ref,
                      out_ref, base_ref):
    L, D = base_ref.shape

    @pl.when(pl.program_id(0) == 0)
    def _compute_base():
        ns = stab_ref.shape[0]
        nc = cyc_ref.shape[0]
        oh_s = (sid_ref[...] == lax.broadcasted_iota(jnp.int32, (L, ns), 1)
                ).astype(jnp.float32)
        oh_c = (cid_ref[...] == lax.broadcasted_iota(jnp.int32, (L, nc), 1)
                ).astype(jnp.float32)
        b = jnp.dot(oh_s, stab_ref[...], preferred_element_type=jnp.float32)
        b += jnp.dot(oh_c, cyc_ref[...], preferred_element_type=jnp.float32)
        base_ref[...] = b + val_ref[0, :][None, :]

    syn = syn_ref[...].astype(jnp.float32)
    diff = val_ref[1, :] - val_ref[0, :]
    out_ref[...] = (base_ref[...][None, :, :]
                    + syn[:, :, None] * diff[None, None, :])


def _tc_scbase_body(syn_ref, base_ref, val_ref, part_ref, out_ref):
    del part_ref
    syn = syn_ref[...].astype(jnp.float32)
    b0 = base_ref[...] + val_ref[0, :][None, :]
    diff = val_ref[1, :] - val_ref[0, :]
    out_ref[...] = b0[None, :, :] + syn[:, :, None] * diff[None, None, :]


def kernel(syndrome, stab_id, cycle_id, stab_table, cycle_table, val_table):
    B, L = syndrome.shape
    D = stab_table.shape[1]
    BB = 32
    B1 = 256
    nb1 = B1 // BB

    sid32 = stab_id.astype(jnp.int32)
    cid32 = cycle_id.astype(jnp.int32)
    syn = syndrome.astype(jnp.int32)

    ns = stab_table.shape[0]
    ns_pad = ((ns + 127) // 128) * 128
    stab_pad = (jnp.pad(stab_table, ((0, ns_pad - ns), (0, 0)))
                if ns_pad != ns else stab_table)

    part = pl.pallas_call(
        _tc_selfbase_body,
        grid=(nb1,),
        in_specs=[
            pl.BlockSpec((L, 1), lambda i: (0, 0)),
            pl.BlockSpec((L, 1), lambda i: (0, 0)),
            pl.BlockSpec((BB, L), lambda i: (i, 0)),
            pl.BlockSpec((ns_pad, D), lambda i: (0, 0)),
            pl.BlockSpec(cycle_table.shape, lambda i: (0, 0)),
            pl.BlockSpec((2, D), lambda i: (0, 0)),
        ],
        out_specs=pl.BlockSpec((BB, L, D), lambda i: (i, 0, 0)),
        out_shape=jax.ShapeDtypeStruct((B, L, D), jnp.float32),
        scratch_shapes=[pltpu.VMEM((L, D), jnp.float32)],
    )(sid32.reshape(L, 1), cid32.reshape(L, 1), syn, stab_pad, cycle_table,
      val_table)

    base = _sc_build_base(sid32, cid32, stab_table, cycle_table)

    return pl.pallas_call(
        _tc_scbase_body,
        grid=((B - B1) // BB,),
        in_specs=[
            pl.BlockSpec((BB, L), lambda i: (i + nb1, 0)),
            pl.BlockSpec((L, D), lambda i: (0, 0)),
            pl.BlockSpec((2, D), lambda i: (0, 0)),
            pl.BlockSpec(memory_space=pl.ANY),
        ],
        out_specs=pl.BlockSpec((BB, L, D), lambda i: (i + nb1, 0, 0)),
        out_shape=jax.ShapeDtypeStruct((B, L, D), jnp.float32),
        input_output_aliases={3: 0},
    )(syn, base, val_table, part)


# speedup vs baseline: 1.0927x; 1.0371x over previous
import functools

import jax
import jax.numpy as jnp
from jax import lax
from jax.experimental import pallas as pl
from jax.experimental.pallas import tpu as pltpu
from jax.experimental.pallas import tpu_sc as plsc

_LANES = 16


def _sc_base_body(toks_per_w, n_used, ids_hbm, comb_hbm, base_hbm,
                  idx_v, rows_v, sem):
    D = rows_v.shape[1]
    wid = lax.axis_index("s") * 2 + lax.axis_index("c")

    @pl.when(wid < n_used)
    def _():
        tok0 = wid * toks_per_w
        pltpu.async_copy(ids_hbm.at[wid], idx_v, sem).wait()
        pltpu.async_copy(comb_hbm.at[idx_v], rows_v, sem).wait()
        for j in range(toks_per_w):
            for k in range(D // _LANES):
                ds = pl.ds(k * _LANES, _LANES)
                rows_v[j, ds] = rows_v[j, ds] + rows_v[toks_per_w + j, ds]
        pltpu.sync_copy(rows_v.at[pl.ds(0, toks_per_w)],
                        base_hbm.at[pl.ds(tok0, toks_per_w)])


def _sc_build_base(stab_id, cycle_id, stab_table, cycle_table):
    L = stab_id.shape[0]
    ns, D = stab_table.shape
    toks_per_w = 8
    n_used = L // toks_per_w
    ids16 = jnp.concatenate(
        [stab_id.reshape(n_used, toks_per_w),
         cycle_id.reshape(n_used, toks_per_w) + ns], axis=1)
    comb = jnp.concatenate([stab_table, cycle_table], axis=0)
    mesh = plsc.VectorSubcoreMesh(core_axis_name="c", subcore_axis_name="s")
    body = functools.partial(_sc_base_body, toks_per_w, n_used)
    return pl.kernel(
        body,
        out_type=jax.ShapeDtypeStruct((L, D), jnp.float32),
        mesh=mesh,
        scratch_types=[
            pltpu.VMEM((2 * toks_per_w,), jnp.int32),
            pltpu.VMEM((2 * toks_per_w, D), jnp.float32),
            pltpu.SemaphoreType.DMA,
        ],
    )(ids16, comb)


def _tc_stream_body(syn_ref, base_ref, val_ref, out_ref):
    syn = syn_ref[...].astype(jnp.float32)
    b0 = base_ref[...] + val_ref[0, :][None, :]
    diff = val_ref[1, :] - val_ref[0, :]
    out_ref[...] = b0[None, :, :] + syn[:, :, None] * diff[None, None, :]


def kernel(syndrome, stab_id, cycle_id, stab_table, cycle_table, val_table):
    B, L = syndrome.shape
    D = stab_table.shape[1]
    BB = 32

    base = _sc_build_base(stab_id.astype(jnp.int32),
                          cycle_id.astype(jnp.int32),
                          stab_table, cycle_table)
    syn = syndrome.astype(jnp.int32)

    return pl.pallas_call(
        _tc_stream_body,
        grid=(B // BB,),
        in_specs=[
            pl.BlockSpec((BB, L), lambda i: (i, 0)),
            pl.BlockSpec((L, D), lambda i: (0, 0)),
            pl.BlockSpec((2, D), lambda i: (0, 0)),
        ],
        out_specs=pl.BlockSpec((BB, L, D), lambda i: (i, 0, 0)),
        out_shape=jax.ShapeDtypeStruct((B, L, D), jnp.float32),
    )(syn, base, val_table)
